# baseline (speedup 1.0000x reference)
import functools

import jax
import jax.numpy as jnp
from jax import lax
from jax.experimental import pallas as pl
from jax.experimental.pallas import tpu as pltpu
from jax.experimental.pallas import tpu_sc as plsc

BATCH = 16384
NFIELD = 26
DIM = 16
NC = 2
NS = 16
NW = NC * NS
PER_W = BATCH // NW
FLAT_PER_W = PER_W * NFIELD
CB = 64
CHUNKS = PER_W // CB
ROWS_PER_CHUNK = CB * NFIELD
DMA_ROWS = 128
DMAS_PER_CHUNK = ROWS_PER_CHUNK // DMA_ROWS


@functools.partial(
    pl.kernel,
    mesh=plsc.VectorSubcoreMesh(core_axis_name="c", subcore_axis_name="s"),
    compiler_params=pltpu.CompilerParams(use_tc_tiling_on_sc=False),
    out_type=jax.ShapeDtypeStruct((BATCH, DIM), jnp.float32),
    scratch_types=[
        pltpu.VMEM((FLAT_PER_W,), jnp.int32),
        pltpu.VMEM((ROWS_PER_CHUNK, DIM), jnp.float32),
        pltpu.VMEM((ROWS_PER_CHUNK, DIM), jnp.float32),
        pltpu.VMEM((PER_W, DIM), jnp.float32),
        pltpu.VMEM((DIM,), jnp.float32),
        pltpu.VMEM((208,), jnp.int32),
        pltpu.SemaphoreType.DMA,
        pltpu.SemaphoreType.DMA,
    ],
)
def _sc_kernel(xf, table, bias, off, out, x_v, rows0, rows1, out_v, bias_v,
               off_v, sem0, sem1):
    wid = lax.axis_index("s") * NC + lax.axis_index("c")
    base = wid * FLAT_PER_W
    pltpu.sync_copy(xf.at[pl.ds(base, FLAT_PER_W)], x_v)
    pltpu.sync_copy(bias, bias_v)
    pltpu.sync_copy(off, off_v)

    def prep(p, carry):
        s = p * 16
        m = lax.rem(p, 13) * 16
        x_v[pl.ds(s, 16)] = x_v[pl.ds(s, 16)] + off_v[pl.ds(m, 16)]
        return carry

    lax.fori_loop(0, FLAT_PER_W // 16, prep, 0)

    bufs = (rows0, rows1)
    sems = (sem0, sem1)

    def fire(c, buf, sem):
        handles = []
        for j in range(DMAS_PER_CHUNK):
            s = c * ROWS_PER_CHUNK + j * DMA_ROWS
            handles.append(
                pltpu.async_copy(
                    table.at[x_v.at[pl.ds(s, DMA_ROWS)]],
                    buf.at[pl.ds(j * DMA_ROWS, DMA_ROWS), :],
                    sem,
                ))
        return handles

    def compute(c, buf):
        def body(b, carry):
            r = b * NFIELD
            acc = bias_v[...]
            for f in range(NFIELD):
                acc = acc + buf[r + f, :]
            out_v[c * CB + b, :] = acc
            return carry

        lax.fori_loop(0, CB, body, 0)

    pending = fire(0, bufs[0], sems[0])
    for c in range(CHUNKS):
        nxt = fire(c + 1, bufs[(c + 1) % 2], sems[(c + 1) % 2]) \
            if c + 1 < CHUNKS else []
        for h in pending:
            h.wait()
        compute(c, bufs[c % 2])
        pending = nxt

    pltpu.sync_copy(out_v, out.at[pl.ds(wid * PER_W, PER_W), :])


def kernel(x, table, bias, offsets):
    xf = x.reshape(-1).astype(jnp.int32)
    off = jnp.tile(offsets.astype(jnp.int32), 208 // NFIELD)
    b16 = bias.reshape(DIM).astype(jnp.float32)
    return _sc_kernel(xf, table, b16, off)


# speedup vs baseline: 1.0054x; 1.0054x over previous
import functools

import jax
import jax.numpy as jnp
from jax import lax
from jax.experimental import pallas as pl
from jax.experimental.pallas import tpu as pltpu
from jax.experimental.pallas import tpu_sc as plsc

BATCH = 16384
NFIELD = 26
DIM = 16
NC = 2
NS = 16
NW = NC * NS
PER_W = BATCH // NW
FLAT_PER_W = PER_W * NFIELD
CB = 64
CHUNKS = PER_W // CB
ROWS_PER_CHUNK = CB * NFIELD
DMA_ROWS = 128
DMAS_PER_CHUNK = ROWS_PER_CHUNK // DMA_ROWS


@functools.partial(
    pl.kernel,
    mesh=plsc.VectorSubcoreMesh(core_axis_name="c", subcore_axis_name="s"),
    compiler_params=pltpu.CompilerParams(use_tc_tiling_on_sc=False),
    out_type=jax.ShapeDtypeStruct((BATCH, DIM), jnp.float32),
    scratch_types=[
        pltpu.VMEM((FLAT_PER_W,), jnp.int32),
        pltpu.VMEM((ROWS_PER_CHUNK, DIM), jnp.float32),
        pltpu.VMEM((ROWS_PER_CHUNK, DIM), jnp.float32),
        pltpu.VMEM((PER_W, DIM), jnp.float32),
        pltpu.VMEM((DIM,), jnp.float32),
        pltpu.VMEM((208,), jnp.int32),
        pltpu.SemaphoreType.DMA,
        pltpu.SemaphoreType.DMA,
    ],
)
def _sc_kernel(xf, table, bias, off, out, x_v, rows0, rows1, out_v, bias_v,
               off_v, sem0, sem1):
    wid = lax.axis_index("s") * NC + lax.axis_index("c")
    base = wid * FLAT_PER_W
    pltpu.sync_copy(xf.at[pl.ds(base, FLAT_PER_W)], x_v)
    pltpu.sync_copy(bias, bias_v)
    pltpu.sync_copy(off, off_v)

    @plsc.parallel_loop(0, FLAT_PER_W // 16)
    def prep(p):
        s = p * 16
        m = lax.rem(p, 13) * 16
        x_v[pl.ds(s, 16)] = x_v[pl.ds(s, 16)] + off_v[pl.ds(m, 16)]

    bufs = (rows0, rows1)
    sems = (sem0, sem1)

    def fire(c, buf, sem):
        handles = []
        for j in range(DMAS_PER_CHUNK):
            s = c * ROWS_PER_CHUNK + j * DMA_ROWS
            handles.append(
                pltpu.async_copy(
                    table.at[x_v.at[pl.ds(s, DMA_ROWS)]],
                    buf.at[pl.ds(j * DMA_ROWS, DMA_ROWS), :],
                    sem,
                ))
        return handles

    def compute(c, buf):
        @plsc.parallel_loop(0, CB)
        def body(b):
            r = b * NFIELD
            accs = [buf[r + f, :] for f in range(4)]
            accs[0] = accs[0] + bias_v[...]
            for f in range(4, NFIELD):
                accs[f % 4] = accs[f % 4] + buf[r + f, :]
            out_v[c * CB + b, :] = (accs[0] + accs[1]) + (accs[2] + accs[3])

    pending = fire(0, bufs[0], sems[0])
    for c in range(CHUNKS):
        nxt = fire(c + 1, bufs[(c + 1) % 2], sems[(c + 1) % 2]) \
            if c + 1 < CHUNKS else []
        for h in pending:
            h.wait()
        compute(c, bufs[c % 2])
        pending = nxt

    pltpu.sync_copy(out_v, out.at[pl.ds(wid * PER_W, PER_W), :])


def kernel(x, table, bias, offsets):
    xf = x.reshape(-1).astype(jnp.int32)
    off = jnp.tile(offsets.astype(jnp.int32), 208 // NFIELD)
    b16 = bias.reshape(DIM).astype(jnp.float32)
    return _sc_kernel(xf, table, b16, off)


# speedup vs baseline: 1.1334x; 1.1273x over previous
import functools

import jax
import jax.numpy as jnp
from jax import lax
from jax.experimental import pallas as pl
from jax.experimental.pallas import tpu as pltpu
from jax.experimental.pallas import tpu_sc as plsc

BATCH = 16384
NFIELD = 26
DIM = 16
TOTAL_ROWS = 2600000
NC = 2
NS = 16
NW = NC * NS
PER_W = BATCH // NW
FLAT_PER_W = PER_W * NFIELD
CB = 64
CHUNKS = PER_W // CB
ROWS_PER_CHUNK = CB * NFIELD
DMA_ROWS = 128
DMAS_PER_CHUNK = ROWS_PER_CHUNK // DMA_ROWS


@functools.partial(
    pl.kernel,
    mesh=plsc.VectorSubcoreMesh(core_axis_name="c", subcore_axis_name="s"),
    compiler_params=pltpu.CompilerParams(use_tc_tiling_on_sc=False),
    out_type=jax.ShapeDtypeStruct((BATCH, DIM), jnp.float32),
    scratch_types=[
        pltpu.VMEM((FLAT_PER_W,), jnp.int32),
        pltpu.VMEM((ROWS_PER_CHUNK, DIM), jnp.float32),
        pltpu.VMEM((ROWS_PER_CHUNK, DIM), jnp.float32),
        pltpu.VMEM((PER_W, DIM), jnp.float32),
        pltpu.VMEM((DIM,), jnp.float32),
        pltpu.VMEM((208,), jnp.int32),
        pltpu.SemaphoreType.DMA,
        pltpu.SemaphoreType.DMA,
    ],
)
def _sc_kernel(xf, table, bias, off, out, x_v, rows0, rows1, out_v, bias_v,
               off_v, sem0, sem1):
    wid = lax.axis_index("s") * NC + lax.axis_index("c")
    base = wid * FLAT_PER_W
    pltpu.sync_copy(xf.at[pl.ds(base, FLAT_PER_W)], x_v)
    pltpu.sync_copy(bias, bias_v)
    pltpu.sync_copy(off, off_v)

    @plsc.parallel_loop(0, FLAT_PER_W // 16)
    def prep(p):
        s = p * 16
        m = lax.rem(p, 13) * 16
        x_v[pl.ds(s, 16)] = x_v[pl.ds(s, 16)] + off_v[pl.ds(m, 16)]

    bufs = (rows0, rows1)
    sems = (sem0, sem1)

    tbl = table

    def fire(c, buf, sem):
        s = c * ROWS_PER_CHUNK
        return [
            pltpu.async_copy(
                tbl.at[x_v.at[pl.ds(s, ROWS_PER_CHUNK)]],
                buf,
                sem,
            )
        ]

    def compute(c, buf):
        @plsc.parallel_loop(0, CB)
        def body(b):
            r = b * NFIELD
            accs = [buf[r + f, :] for f in range(4)]
            accs[0] = accs[0] + bias_v[...]
            for f in range(4, NFIELD):
                accs[f % 4] = accs[f % 4] + buf[r + f, :]
            out_v[c * CB + b, :] = (accs[0] + accs[1]) + (accs[2] + accs[3])

    pending = fire(0, bufs[0], sems[0])
    for c in range(CHUNKS):
        nxt = fire(c + 1, bufs[(c + 1) % 2], sems[(c + 1) % 2]) \
            if c + 1 < CHUNKS else []
        for h in pending:
            h.wait()
        compute(c, bufs[c % 2])
        pending = nxt

    pltpu.sync_copy(out_v, out.at[pl.ds(wid * PER_W, PER_W), :])


RN = 1024
_FULL_RUNS = TOTAL_ROWS // RN
_TAIL = TOTAL_ROWS - _FULL_RUNS * RN
_RPW = (_FULL_RUNS + NW - 1) // NW


@functools.partial(
    pl.kernel,
    mesh=plsc.VectorSubcoreMesh(core_axis_name="c", subcore_axis_name="s"),
    compiler_params=pltpu.CompilerParams(
        use_tc_tiling_on_sc=True, needs_layout_passes=False),
    out_type=jax.ShapeDtypeStruct((TOTAL_ROWS // 8, DIM * 8), jnp.float32),
    scratch_types=[
        pltpu.VMEM((DIM * RN,), jnp.float32),
        pltpu.VMEM((DIM * RN,), jnp.float32),
        pltpu.VMEM((RN // 8, DIM * 8), jnp.float32),
        pltpu.SemaphoreType.DMA,
        pltpu.SemaphoreType.DMA,
    ],
)
def _sc_detile(tblt, tail8, out, strips0, strips1, rows_v, sem0, sem1):
    wid = lax.axis_index("s") * NC + lax.axis_index("c")
    lane = lax.iota(jnp.int32, 16)
    gbase = lane * RN

    def run_of(k):
        return jnp.minimum(wid + k * NW, _FULL_RUNS - 1)

    def fire(k, buf, sem):
        r0 = run_of(k) * RN
        for c in range(DIM):
            pltpu.async_copy(
                tblt.at[c, pl.ds(r0, RN)], buf.at[pl.ds(c * RN, RN)], sem)

    def drain(buf, sem):
        for c in range(DIM):
            pltpu.make_async_copy(
                tblt.at[0, pl.ds(0, RN)], buf.at[pl.ds(c * RN, RN)], sem
            ).wait()

    def transpose_run(k, buf):
        @plsc.parallel_loop(0, RN)
        def row(i):
            r = plsc.load_gather(buf, [gbase + i])
            rows_v[lax.shift_right_logical(i, 3),
                   pl.ds(lax.mul(lax.rem(i, 8), DIM), DIM)] = r

        pltpu.sync_copy(
            rows_v, out.at[pl.ds(run_of(k) * (RN // 8), RN // 8), :])

    fire(0, strips0, sem0)

    def pair(j, carry):
        k0 = j * 2
        drain(strips0, sem0)
        fire(k0 + 1, strips1, sem1)
        transpose_run(k0, strips0)
        drain(strips1, sem1)
        fire(k0 + 2, strips0, sem0)
        transpose_run(k0 + 1, strips1)
        return carry

    lax.fori_loop(0, _RPW // 2, pair, 0)
    drain(strips0, sem0)

    @pl.when(wid == 0)
    def _tail():
        pltpu.sync_copy(
            tail8, out.at[pl.ds(_FULL_RUNS * RN // 8, _TAIL // 8), :])


def kernel(x, table, bias, offsets):
    xf = x.reshape(-1).astype(jnp.int32)
    off = jnp.tile(offsets.astype(jnp.int32), 208 // NFIELD)
    b16 = bias.reshape(DIM).astype(jnp.float32)
    tail8 = jax.lax.slice(
        table, (_FULL_RUNS * RN, 0), (TOTAL_ROWS, DIM)).reshape(
            _TAIL // 8, DIM * 8)
    tbl_lin = _sc_detile(table.T, tail8).reshape(TOTAL_ROWS, DIM)
    return _sc_kernel(xf, tbl_lin, b16, off)


# speedup vs baseline: 1.8795x; 1.6582x over previous
import functools

import jax
import jax.numpy as jnp
from jax import lax
from jax.experimental import pallas as pl
from jax.experimental.pallas import tpu as pltpu
from jax.experimental.pallas import tpu_sc as plsc

BATCH = 16384
NFIELD = 26
DIM = 16
TOTAL_ROWS = 2600000
NC = 2
NS = 16
NW = NC * NS
PER_W = BATCH // NW
FLAT_PER_W = PER_W * NFIELD
CB = 64
CHUNKS = PER_W // CB
ROWS_PER_CHUNK = CB * NFIELD
DMA_ROWS = 128
DMAS_PER_CHUNK = ROWS_PER_CHUNK // DMA_ROWS


@functools.partial(
    pl.kernel,
    mesh=plsc.VectorSubcoreMesh(core_axis_name="c", subcore_axis_name="s"),
    compiler_params=pltpu.CompilerParams(use_tc_tiling_on_sc=False),
    out_type=jax.ShapeDtypeStruct((BATCH, DIM), jnp.float32),
    scratch_types=[
        pltpu.VMEM((FLAT_PER_W,), jnp.int32),
        pltpu.VMEM((ROWS_PER_CHUNK, DIM), jnp.float32),
        pltpu.VMEM((ROWS_PER_CHUNK, DIM), jnp.float32),
        pltpu.VMEM((PER_W, DIM), jnp.float32),
        pltpu.VMEM((DIM,), jnp.float32),
        pltpu.VMEM((208,), jnp.int32),
        pltpu.SemaphoreType.DMA,
        pltpu.SemaphoreType.DMA,
    ],
)
def _sc_kernel(xf, table, bias, off, out, x_v, rows0, rows1, out_v, bias_v,
               off_v, sem0, sem1):
    wid = lax.axis_index("s") * NC + lax.axis_index("c")
    base = wid * FLAT_PER_W
    pltpu.sync_copy(xf.at[pl.ds(base, FLAT_PER_W)], x_v)
    pltpu.sync_copy(bias, bias_v)
    pltpu.sync_copy(off, off_v)

    @plsc.parallel_loop(0, FLAT_PER_W // 16)
    def prep(p):
        s = p * 16
        m = lax.rem(p, 13) * 16
        x_v[pl.ds(s, 16)] = x_v[pl.ds(s, 16)] + off_v[pl.ds(m, 16)]

    bufs = (rows0, rows1)
    sems = (sem0, sem1)

    tbl = table

    def fire(c, buf, sem):
        s = c * ROWS_PER_CHUNK
        return [
            pltpu.async_copy(
                tbl.at[x_v.at[pl.ds(s, ROWS_PER_CHUNK)]],
                buf,
                sem,
            )
        ]

    def compute(c, buf):
        @plsc.parallel_loop(0, CB)
        def body(b):
            r = b * NFIELD
            accs = [buf[r + f, :] for f in range(4)]
            accs[0] = accs[0] + bias_v[...]
            for f in range(4, NFIELD):
                accs[f % 4] = accs[f % 4] + buf[r + f, :]
            out_v[c * CB + b, :] = (accs[0] + accs[1]) + (accs[2] + accs[3])

    pending = fire(0, bufs[0], sems[0])
    for c in range(CHUNKS):
        nxt = fire(c + 1, bufs[(c + 1) % 2], sems[(c + 1) % 2]) \
            if c + 1 < CHUNKS else []
        for h in pending:
            h.wait()
        compute(c, bufs[c % 2])
        pending = nxt

    pltpu.sync_copy(out_v, out.at[pl.ds(wid * PER_W, PER_W), :])


RN = 1024
_FULL_RUNS = TOTAL_ROWS // RN
_TAIL = TOTAL_ROWS - _FULL_RUNS * RN
_RPW = (_FULL_RUNS + NW - 1) // NW


@functools.partial(
    pl.kernel,
    mesh=plsc.VectorSubcoreMesh(core_axis_name="c", subcore_axis_name="s"),
    compiler_params=pltpu.CompilerParams(
        use_tc_tiling_on_sc=True, needs_layout_passes=False),
    out_type=jax.ShapeDtypeStruct((TOTAL_ROWS * DIM,), jnp.float32),
    scratch_types=[
        pltpu.VMEM((DIM * RN,), jnp.float32),
        pltpu.VMEM((DIM * RN,), jnp.float32),
        pltpu.VMEM((RN * DIM,), jnp.float32),
        pltpu.SemaphoreType.DMA,
        pltpu.SemaphoreType.DMA,
    ],
)
def _sc_detile(tblt, tail8, out, strips0, strips1, rows_v, sem0, sem1):
    wid = lax.axis_index("s") * NC + lax.axis_index("c")
    lane = lax.iota(jnp.int32, 16)
    gbase = lane * RN

    def run_of(k):
        return jnp.minimum(wid + k * NW, _FULL_RUNS - 1)

    def fire(k, buf, sem):
        r0 = run_of(k) * RN
        for c in range(DIM):
            pltpu.async_copy(
                tblt.at[c, pl.ds(r0, RN)], buf.at[pl.ds(c * RN, RN)], sem)

    def drain(buf, sem):
        for c in range(DIM):
            pltpu.make_async_copy(
                tblt.at[0, pl.ds(0, RN)], buf.at[pl.ds(c * RN, RN)], sem
            ).wait()

    def transpose_run(k, buf):
        @plsc.parallel_loop(0, RN, unroll=8)
        def row(i):
            r = plsc.load_gather(buf, [gbase + i])
            rows_v[pl.ds(i * DIM, DIM)] = r

        pltpu.sync_copy(
            rows_v, out.at[pl.ds(run_of(k) * (RN * DIM), RN * DIM)])

    fire(0, strips0, sem0)

    def pair(j, carry):
        k0 = j * 2
        drain(strips0, sem0)
        fire(k0 + 1, strips1, sem1)
        transpose_run(k0, strips0)
        drain(strips1, sem1)
        fire(k0 + 2, strips0, sem0)
        transpose_run(k0 + 1, strips1)
        return carry

    lax.fori_loop(0, _RPW // 2, pair, 0)
    drain(strips0, sem0)

    @pl.when(wid == 0)
    def _tail():
        pltpu.sync_copy(
            tail8, out.at[pl.ds(_FULL_RUNS * RN * DIM, _TAIL * DIM)])


def kernel(x, table, bias, offsets):
    xf = x.reshape(-1).astype(jnp.int32)
    off = jnp.tile(offsets.astype(jnp.int32), 208 // NFIELD)
    b16 = bias.reshape(DIM).astype(jnp.float32)
    tail8 = jax.lax.slice(
        table, (_FULL_RUNS * RN, 0), (TOTAL_ROWS, DIM)).reshape(_TAIL * DIM)
    tbl_lin = _sc_detile(table.T, tail8).reshape(TOTAL_ROWS, DIM)
    return _sc_kernel(xf, tbl_lin, b16, off)


# speedup vs baseline: 2.0507x; 1.0911x over previous
import functools

import jax
import jax.numpy as jnp
from jax import lax
from jax.experimental import pallas as pl
from jax.experimental.pallas import tpu as pltpu
from jax.experimental.pallas import tpu_sc as plsc

BATCH = 16384
NFIELD = 26
DIM = 16
TOTAL_ROWS = 2600000
NC = 2
NS = 16
NW = NC * NS
PER_W = BATCH // NW
FLAT_PER_W = PER_W * NFIELD
CB = 64
CHUNKS = PER_W // CB
ROWS_PER_CHUNK = CB * NFIELD
DMA_ROWS = 128
DMAS_PER_CHUNK = ROWS_PER_CHUNK // DMA_ROWS


@functools.partial(
    pl.kernel,
    mesh=plsc.VectorSubcoreMesh(core_axis_name="c", subcore_axis_name="s"),
    compiler_params=pltpu.CompilerParams(use_tc_tiling_on_sc=False),
    out_type=jax.ShapeDtypeStruct((BATCH, DIM), jnp.float32),
    scratch_types=[
        pltpu.VMEM((FLAT_PER_W,), jnp.int32),
        pltpu.VMEM((ROWS_PER_CHUNK, DIM), jnp.float32),
        pltpu.VMEM((ROWS_PER_CHUNK, DIM), jnp.float32),
        pltpu.VMEM((PER_W, DIM), jnp.float32),
        pltpu.VMEM((DIM,), jnp.float32),
        pltpu.VMEM((208,), jnp.int32),
        pltpu.SemaphoreType.DMA,
        pltpu.SemaphoreType.DMA,
    ],
)
def _sc_kernel(xf, table, bias, off, out, x_v, rows0, rows1, out_v, bias_v,
               off_v, sem0, sem1):
    wid = lax.axis_index("s") * NC + lax.axis_index("c")
    base = wid * FLAT_PER_W
    pltpu.sync_copy(xf.at[pl.ds(base, FLAT_PER_W)], x_v)
    pltpu.sync_copy(bias, bias_v)
    pltpu.sync_copy(off, off_v)

    @plsc.parallel_loop(0, FLAT_PER_W // 16)
    def prep(p):
        s = p * 16
        m = lax.rem(p, 13) * 16
        x_v[pl.ds(s, 16)] = x_v[pl.ds(s, 16)] + off_v[pl.ds(m, 16)]

    bufs = (rows0, rows1)
    sems = (sem0, sem1)

    tbl = table

    def fire(c, buf, sem):
        s = c * ROWS_PER_CHUNK
        return [
            pltpu.async_copy(
                tbl.at[x_v.at[pl.ds(s, ROWS_PER_CHUNK)]],
                buf,
                sem,
            )
        ]

    def compute(c, buf):
        @plsc.parallel_loop(0, CB)
        def body(b):
            r = b * NFIELD
            accs = [buf[r + f, :] for f in range(4)]
            accs[0] = accs[0] + bias_v[...]
            for f in range(4, NFIELD):
                accs[f % 4] = accs[f % 4] + buf[r + f, :]
            out_v[c * CB + b, :] = (accs[0] + accs[1]) + (accs[2] + accs[3])

    pending = fire(0, bufs[0], sems[0])
    for c in range(CHUNKS):
        nxt = fire(c + 1, bufs[(c + 1) % 2], sems[(c + 1) % 2]) \
            if c + 1 < CHUNKS else []
        for h in pending:
            h.wait()
        compute(c, bufs[c % 2])
        pending = nxt

    pltpu.sync_copy(out_v, out.at[pl.ds(wid * PER_W, PER_W), :])


RN = 1024
_FULL_RUNS = TOTAL_ROWS // RN
_TAIL = TOTAL_ROWS - _FULL_RUNS * RN
_RPW = (_FULL_RUNS + NW - 1) // NW


@functools.partial(
    pl.kernel,
    mesh=plsc.VectorSubcoreMesh(core_axis_name="c", subcore_axis_name="s"),
    compiler_params=pltpu.CompilerParams(
        use_tc_tiling_on_sc=True, needs_layout_passes=False),
    out_type=jax.ShapeDtypeStruct((TOTAL_ROWS * DIM,), jnp.float32),
    scratch_types=[
        pltpu.VMEM((DIM * RN,), jnp.float32),
        pltpu.VMEM((DIM * RN,), jnp.float32),
        pltpu.VMEM((RN * DIM,), jnp.float32),
        pltpu.VMEM((RN * DIM,), jnp.float32),
        pltpu.SemaphoreType.DMA,
        pltpu.SemaphoreType.DMA,
        pltpu.SemaphoreType.DMA,
        pltpu.SemaphoreType.DMA,
    ],
)
def _sc_detile(tblt, tail8, out, strips0, strips1, rows0, rows1,
               sem0, sem1, osem0, osem1):
    wid = lax.axis_index("s") * NC + lax.axis_index("c")
    lane = lax.iota(jnp.int32, 16)
    gbase = lane * RN

    def run_of(k):
        return jnp.minimum(wid + k * NW, _FULL_RUNS - 1)

    def fire(k, buf, sem):
        r0 = run_of(k) * RN
        for c in range(DIM):
            pltpu.async_copy(
                tblt.at[c, pl.ds(r0, RN)], buf.at[pl.ds(c * RN, RN)], sem)

    def drain(buf, sem):
        for c in range(DIM):
            pltpu.make_async_copy(
                tblt.at[0, pl.ds(0, RN)], buf.at[pl.ds(c * RN, RN)], sem
            ).wait()

    def owait(rv, osem):
        pltpu.make_async_copy(
            out.at[pl.ds(0, RN * DIM)], rv, osem).wait()

    def transpose_run(k, buf, rv, osem, j):
        @pl.when(j > 0)
        def _():
            owait(rv, osem)

        @plsc.parallel_loop(0, RN, unroll=8)
        def row(i):
            r = plsc.load_gather(buf, [gbase + i])
            rv[pl.ds(i * DIM, DIM)] = r

        pltpu.async_copy(
            rv, out.at[pl.ds(run_of(k) * (RN * DIM), RN * DIM)], osem)

    fire(0, strips0, sem0)

    def pair(j, carry):
        k0 = j * 2
        drain(strips0, sem0)
        fire(k0 + 1, strips1, sem1)
        transpose_run(k0, strips0, rows0, osem0, j)
        drain(strips1, sem1)
        fire(k0 + 2, strips0, sem0)
        transpose_run(k0 + 1, strips1, rows1, osem1, j)
        return carry

    lax.fori_loop(0, _RPW // 2, pair, 0)
    drain(strips0, sem0)
    owait(rows0, osem0)
    owait(rows1, osem1)

    @pl.when(wid == 0)
    def _tail():
        pltpu.sync_copy(
            tail8, out.at[pl.ds(_FULL_RUNS * RN * DIM, _TAIL * DIM)])


def kernel(x, table, bias, offsets):
    xf = x.reshape(-1).astype(jnp.int32)
    off = jnp.tile(offsets.astype(jnp.int32), 208 // NFIELD)
    b16 = bias.reshape(DIM).astype(jnp.float32)
    tail8 = jax.lax.slice(
        table, (_FULL_RUNS * RN, 0), (TOTAL_ROWS, DIM)).reshape(_TAIL * DIM)
    tbl_lin = _sc_detile(table.T, tail8).reshape(TOTAL_ROWS, DIM)
    return _sc_kernel(xf, tbl_lin, b16, off)
